# baseline (speedup 1.0000x reference)
import jax
import jax.numpy as jnp
from jax.experimental import pallas as pl
from jax.experimental.pallas import tpu as pltpu

_CHUNK_ROWS = (1024, 1024, 1024, 512, 256, 256)


def kernel(x, queue):
    del queue
    B, D = x.shape
    n = len(_CHUNK_ROWS)
    offs = [0]
    for c in _CHUNK_ROWS:
        offs.append(offs[-1] + c)
    assert offs[-1] == B

    def body(x_hbm, o_hbm, buf, in_sems, out_sems):
        reads = [
            pltpu.make_async_copy(
                x_hbm.at[pl.ds(offs[i], _CHUNK_ROWS[i])],
                buf.at[pl.ds(offs[i], _CHUNK_ROWS[i])],
                in_sems.at[i])
            for i in range(n)
        ]
        writes = [
            pltpu.make_async_copy(
                buf.at[pl.ds(offs[i], _CHUNK_ROWS[i])],
                o_hbm.at[pl.ds(offs[i], _CHUNK_ROWS[i])],
                out_sems.at[i])
            for i in range(n)
        ]
        for r in reads:
            r.start()
        for r, w in zip(reads, writes):
            r.wait()
            w.start()
        for w in writes:
            w.wait()

    return pl.pallas_call(
        body,
        in_specs=[pl.BlockSpec(memory_space=pl.ANY)],
        out_specs=pl.BlockSpec(memory_space=pl.ANY),
        out_shape=jax.ShapeDtypeStruct((B, D), x.dtype),
        scratch_shapes=[
            pltpu.VMEM((B, D), x.dtype),
            pltpu.SemaphoreType.DMA((n,)),
            pltpu.SemaphoreType.DMA((n,)),
        ],
    )(x)
